# baseline (speedup 1.0000x reference)
import functools

import jax
import jax.numpy as jnp
from jax import lax
from jax.experimental import pallas as pl
from jax.experimental.pallas import tpu as pltpu
from jax.experimental.pallas import tpu_sc as plsc


def _make_sc_gather(row0, rows, m, table_n):
    info = plsc.get_sparse_core_info()
    nc, ns, nl = info.num_cores, info.num_subcores, info.num_lanes
    nw = nc * ns
    assert rows % nw == 0 and m % nl == 0
    rows_per_w = rows // nw
    chunks_per_row = m // nl
    table_pad = ((table_n + nl - 1) // nl) * nl
    mesh = plsc.VectorSubcoreMesh(core_axis_name="c", subcore_axis_name="s")

    @functools.partial(
        pl.kernel,
        mesh=mesh,
        out_type=jax.ShapeDtypeStruct((rows, m), jnp.float32),
        scratch_types=[
            pltpu.VMEM((table_pad,), jnp.float32),
            pltpu.VMEM((rows_per_w, m), jnp.int32),
            pltpu.VMEM((rows_per_w, m), jnp.float32),
        ],
        compiler_params=pltpu.CompilerParams(needs_layout_passes=False),
    )
    def sc_gather(table_h, idx_h, bias_h, table_v, idx_v, bias_v):
        wid = lax.axis_index("s") * nc + lax.axis_index("c")
        base = wid * rows_per_w
        pltpu.sync_copy(table_h, table_v.at[pl.ds(0, table_n)])
        pltpu.sync_copy(idx_h.at[pl.ds(row0 + base, rows_per_w), :], idx_v)

        @plsc.parallel_loop(0, rows_per_w)
        def _(r):
            for c in range(chunks_per_row):
                iv = idx_v[r, pl.ds(c * nl, nl)]
                bias_v[r, pl.ds(c * nl, nl)] = plsc.load_gather(table_v, [iv])

        pltpu.sync_copy(bias_v, bias_h.at[pl.ds(base, rows_per_w), :])

    return sc_gather


def _tc_add_first_body(a_ref, b_ref, o_ref):
    o_ref[...] = a_ref[...] + b_ref[...][None]


def _tc_add_next_body(p_ref, a_ref, b_ref, o_ref):
    del p_ref
    o_ref[...] = a_ref[...] + b_ref[...][None]


def _tc_add_chunk(a3, bias_chunk, chunk_idx, prev, head_block=2):
    nh, n, m = a3.shape
    rows = bias_chunk.shape[0]
    grid = (nh // head_block,)
    a_spec = pl.BlockSpec((head_block, rows, m), lambda h: (h, chunk_idx, 0))
    b_spec = pl.BlockSpec((rows, m), lambda h: (0, 0))
    out_spec = pl.BlockSpec((head_block, rows, m), lambda h: (h, chunk_idx, 0))
    out_shape = jax.ShapeDtypeStruct((nh, n, m), a3.dtype)
    if prev is None:
        return pl.pallas_call(
            _tc_add_first_body,
            grid=grid,
            in_specs=[a_spec, b_spec],
            out_specs=out_spec,
            out_shape=out_shape,
        )(a3, bias_chunk)
    prev_spec = pl.BlockSpec((1, 8, 128), lambda h: (0, 0, 0))
    return pl.pallas_call(
        _tc_add_next_body,
        grid=grid,
        in_specs=[prev_spec, a_spec, b_spec],
        out_specs=out_spec,
        out_shape=out_shape,
        input_output_aliases={0: 0},
    )(prev, a3, bias_chunk)


def kernel(attn_weights, relative_position_index, relative_position_bias_table):
    n, m = relative_position_index.shape
    _, nh, _, _ = attn_weights.shape
    table_flat = relative_position_bias_table.reshape(-1)
    a3 = attn_weights.reshape(nh, n, m)

    n_chunks = 2
    rows = n // n_chunks
    biases = [
        _make_sc_gather(c * rows, rows, m, table_flat.shape[0])(
            table_flat, relative_position_index)
        for c in range(n_chunks)
    ]
    out = None
    for c in range(n_chunks):
        out = _tc_add_chunk(a3, biases[c], c, out)
    return out.reshape(attn_weights.shape)


# speedup vs baseline: 1.0184x; 1.0184x over previous
import functools

import jax
import jax.numpy as jnp
from jax import lax
from jax.experimental import pallas as pl
from jax.experimental.pallas import tpu as pltpu
from jax.experimental.pallas import tpu_sc as plsc


def _make_sc_gather(n, m, table_n):
    info = plsc.get_sparse_core_info()
    nc, ns, nl = info.num_cores, info.num_subcores, info.num_lanes
    nw = nc * ns
    assert n % nw == 0 and m % nl == 0
    rows_per_w = n // nw
    chunks_per_row = m // nl
    table_pad = ((table_n + nl - 1) // nl) * nl
    mesh = plsc.VectorSubcoreMesh(core_axis_name="c", subcore_axis_name="s")

    n_sub = 4
    assert rows_per_w % n_sub == 0
    sub_rows = rows_per_w // n_sub

    @functools.partial(
        pl.kernel,
        mesh=mesh,
        out_type=jax.ShapeDtypeStruct((n, m), jnp.float32),
        scratch_types=[
            pltpu.VMEM((table_pad,), jnp.float32),
            pltpu.VMEM((rows_per_w, m), jnp.int32),
            pltpu.VMEM((rows_per_w, m), jnp.float32),
            [pltpu.SemaphoreType.DMA] * n_sub,
            [pltpu.SemaphoreType.DMA] * n_sub,
        ],
        compiler_params=pltpu.CompilerParams(needs_layout_passes=False),
    )
    def sc_gather(table_h, idx_h, bias_h, table_v, idx_v, bias_v,
                  in_sems, out_sems):
        wid = lax.axis_index("s") * nc + lax.axis_index("c")
        base = wid * rows_per_w
        in_copies = [
            pltpu.async_copy(
                idx_h.at[pl.ds(base + s * sub_rows, sub_rows), :],
                idx_v.at[pl.ds(s * sub_rows, sub_rows), :],
                in_sems[s],
            )
            for s in range(n_sub)
        ]
        pltpu.sync_copy(table_h, table_v.at[pl.ds(0, table_n)])

        out_copies = []
        for s in range(n_sub):
            in_copies[s].wait()

            @plsc.parallel_loop(0, sub_rows)
            def _(r, _s=s):
                row = _s * sub_rows + r
                for c in range(chunks_per_row):
                    iv = idx_v[row, pl.ds(c * nl, nl)]
                    bias_v[row, pl.ds(c * nl, nl)] = plsc.load_gather(
                        table_v, [iv])

            out_copies.append(pltpu.async_copy(
                bias_v.at[pl.ds(s * sub_rows, sub_rows), :],
                bias_h.at[pl.ds(base + s * sub_rows, sub_rows), :],
                out_sems[s],
            ))
        for cp in out_copies:
            cp.wait()

    return sc_gather


def _tc_add_body(a_ref, b_ref, o_ref):
    o_ref[...] = a_ref[...] + b_ref[...][None]


def _tc_add(attn, bias2d, head_block=2):
    _, nh, n, m = attn.shape
    a3 = attn.reshape(nh, n, m)
    grid = (nh // head_block,)
    out = pl.pallas_call(
        _tc_add_body,
        grid=grid,
        in_specs=[
            pl.BlockSpec((head_block, n, m), lambda h: (h, 0, 0)),
            pl.BlockSpec((n, m), lambda h: (0, 0)),
        ],
        out_specs=pl.BlockSpec((head_block, n, m), lambda h: (h, 0, 0)),
        out_shape=jax.ShapeDtypeStruct(a3.shape, a3.dtype),
    )(a3, bias2d)
    return out.reshape(attn.shape)


def kernel(attn_weights, relative_position_index, relative_position_bias_table):
    n, m = relative_position_index.shape
    table_flat = relative_position_bias_table.reshape(-1)

    bias2d = _make_sc_gather(n, m, table_flat.shape[0])(
        table_flat, relative_position_index)
    return _tc_add(attn_weights, bias2d)


# speedup vs baseline: 1.0728x; 1.0534x over previous
import functools

import jax
import jax.numpy as jnp
from jax import lax
from jax.experimental import pallas as pl
from jax.experimental.pallas import tpu as pltpu
from jax.experimental.pallas import tpu_sc as plsc


def _make_sc_gather(n, m, table_n):
    info = plsc.get_sparse_core_info()
    nc, ns, nl = info.num_cores, info.num_subcores, info.num_lanes
    nw = nc * ns
    assert n % nw == 0 and m % nl == 0
    rows_per_w = n // nw
    chunks_per_row = m // nl
    table_pad = ((table_n + nl - 1) // nl) * nl
    mesh = plsc.VectorSubcoreMesh(core_axis_name="c", subcore_axis_name="s")

    @functools.partial(
        pl.kernel,
        mesh=mesh,
        out_type=jax.ShapeDtypeStruct((n, m), jnp.float32),
        scratch_types=[
            pltpu.VMEM((table_pad,), jnp.float32),
            pltpu.VMEM((rows_per_w, m), jnp.int32),
            pltpu.VMEM((rows_per_w, m), jnp.float32),
        ],
        compiler_params=pltpu.CompilerParams(needs_layout_passes=False),
    )
    def sc_gather(table_h, idx_h, bias_h, table_v, idx_v, bias_v):
        wid = lax.axis_index("s") * nc + lax.axis_index("c")
        base = wid * rows_per_w
        pltpu.sync_copy(table_h, table_v.at[pl.ds(0, table_n)])
        pltpu.sync_copy(idx_h.at[pl.ds(base, rows_per_w), :], idx_v)

        @plsc.parallel_loop(0, rows_per_w)
        def _(r):
            for c in range(chunks_per_row):
                iv = idx_v[r, pl.ds(c * nl, nl)]
                bias_v[r, pl.ds(c * nl, nl)] = plsc.load_gather(table_v, [iv])

        pltpu.sync_copy(bias_v, bias_h.at[pl.ds(base, rows_per_w), :])

    return sc_gather


def _tc_add_body(a_ref, b_ref, o_ref):
    o_ref[...] = a_ref[...] + b_ref[...][None]


def _tc_add(attn, bias2d, head_block=2):
    _, nh, n, m = attn.shape
    a3 = attn.reshape(nh, n, m)
    grid = (nh // head_block,)
    out = pl.pallas_call(
        _tc_add_body,
        grid=grid,
        in_specs=[
            pl.BlockSpec((head_block, n, m), lambda h: (h, 0, 0)),
            pl.BlockSpec((n, m), lambda h: (0, 0)),
        ],
        out_specs=pl.BlockSpec((head_block, n, m), lambda h: (h, 0, 0)),
        out_shape=jax.ShapeDtypeStruct(a3.shape, a3.dtype),
    )(a3, bias2d)
    return out.reshape(attn.shape)


def kernel(attn_weights, relative_position_index, relative_position_bias_table):
    n, m = relative_position_index.shape
    table_flat = relative_position_bias_table.reshape(-1)

    bias2d = _make_sc_gather(n, m, table_flat.shape[0])(
        table_flat, relative_position_index)
    return _tc_add(attn_weights, bias2d)
